# Initial kernel scaffold; baseline (speedup 1.0000x reference)
import jax
import jax.numpy as jnp
from jax.experimental import pallas as pl


def kernel(vertices, faces, uv, uvfaces, uvmap):
    raise NotImplementedError("write your pallas kernel here")


# speedup vs baseline: 2398.1788x; 2398.1788x over previous
import dataclasses
import functools

import jax
import jax.numpy as jnp
from jax import lax
from jax.experimental import pallas as pl
from jax.experimental.pallas import tpu as pltpu
from jax.experimental.pallas import tpu_sc as plsc

SIZE = 256
NTRI = 256
ROWS_PER_TILE = 8
NTILES = SIZE // ROWS_PER_TILE
NWORKERS = 32
PIX_PER_W = SIZE * SIZE // NWORKERS
CHUNK = 16


def _raster_body(consts_ref, px_ref, py_ref, zmin_ref,
                 win_ref, w1_ref, w2_ref, a_ref):
    pxb = jnp.broadcast_to(px_ref[...], (ROWS_PER_TILE, SIZE))
    pyb = jnp.broadcast_to(py_ref[...], (ROWS_PER_TILE, SIZE))
    zmin = zmin_ref[0, 0]
    zbuf0 = jnp.full((ROWS_PER_TILE, SIZE), zmin, jnp.float32)
    win0 = jnp.full((ROWS_PER_TILE, SIZE), -1, jnp.int32)
    w10 = jnp.zeros((ROWS_PER_TILE, SIZE), jnp.float32)
    w20 = jnp.zeros((ROWS_PER_TILE, SIZE), jnp.float32)

    def body(i, carry):
        zbuf, win, w1s, w2s = carry
        Bx = consts_ref[0, i]
        By = consts_ref[1, i]
        dABy = consts_ref[2, i]
        dABx = consts_ref[3, i]
        Cx = consts_ref[4, i]
        Cy = consts_ref[5, i]
        dBCy = consts_ref[6, i]
        dBCx = consts_ref[7, i]
        Ax = consts_ref[8, i]
        Ay = consts_ref[9, i]
        dCAy = consts_ref[10, i]
        dCAx = consts_ref[11, i]
        Az = consts_ref[12, i]
        Bz = consts_ref[13, i]
        Cz = consts_ref[14, i]
        invw = consts_ref[15, i]
        pAB = (pxb - Bx) * dABy - (pyb - By) * dABx
        pCB = (pxb - Cx) * dBCy - (pyb - Cy) * dBCx
        pCA = (pxb - Ax) * dCAy - (pyb - Ay) * dCAx
        inside = (jnp.maximum(pAB, 0.0) * jnp.maximum(pCB, 0.0)
                  * jnp.maximum(pCA, 0.0)) > 0
        w1 = pCB * invw
        w2 = pCA * invw
        w3 = 1.0 - w1 - w2
        z = (w1 * Az + w2 * Bz) + w3 * Cz
        mask = inside & (z >= zbuf)
        zbuf = jnp.where(mask, z, zbuf)
        win = jnp.where(mask, i, win)
        w1s = jnp.where(mask, w1, w1s)
        w2s = jnp.where(mask, w2, w2s)
        return (zbuf, win, w1s, w2s)

    zbuf, win, w1s, w2s = lax.fori_loop(0, NTRI, body,
                                        (zbuf0, win0, w10, w20))
    win_ref[...] = win
    w1_ref[...] = w1s
    w2_ref[...] = w2s
    a_ref[...] = jnp.where(win >= 0, 1.0, 0.0).astype(jnp.float32)


def _rasterize(consts, px, py, zmin):
    grid = (NTILES,)
    out = pl.pallas_call(
        _raster_body,
        grid=grid,
        in_specs=[
            pl.BlockSpec(memory_space=pltpu.SMEM),
            pl.BlockSpec((1, SIZE), lambda i: (0, 0)),
            pl.BlockSpec((ROWS_PER_TILE, 1), lambda i: (i, 0)),
            pl.BlockSpec(memory_space=pltpu.SMEM),
        ],
        out_specs=[
            pl.BlockSpec((ROWS_PER_TILE, SIZE), lambda i: (i, 0)),
            pl.BlockSpec((ROWS_PER_TILE, SIZE), lambda i: (i, 0)),
            pl.BlockSpec((ROWS_PER_TILE, SIZE), lambda i: (i, 0)),
            pl.BlockSpec((ROWS_PER_TILE, SIZE), lambda i: (i, 0)),
        ],
        out_shape=[
            jax.ShapeDtypeStruct((SIZE, SIZE), jnp.int32),
            jax.ShapeDtypeStruct((SIZE, SIZE), jnp.float32),
            jax.ShapeDtypeStruct((SIZE, SIZE), jnp.float32),
            jax.ShapeDtypeStruct((SIZE, SIZE), jnp.float32),
        ],
    )(consts, px, py, zmin)
    return out


def _sampler_kernel(win_hbm, w1_hbm, w2_hbm, uvt_hbm, tex_hbm,
                    r_hbm, g_hbm, b_hbm,
                    win_v, w1_v, w2_v, uvt_v,
                    xi0_v, xi1_v, yi0_v, yi1_v,
                    w00_v, w10_v, w01_v, w11_v, tex_v, out_v, sem):
    wid = lax.axis_index("c") * 16 + lax.axis_index("s")
    base = wid * PIX_PER_W

    pltpu.sync_copy(win_hbm.at[pl.ds(base, PIX_PER_W)], win_v)
    pltpu.sync_copy(w1_hbm.at[pl.ds(base, PIX_PER_W)], w1_v)
    pltpu.sync_copy(w2_hbm.at[pl.ds(base, PIX_PER_W)], w2_v)
    pltpu.sync_copy(uvt_hbm, uvt_v)

    @pl.loop(0, PIX_PER_W // CHUNK)
    def _(k):
        sl = pl.ds(k * CHUNK, CHUNK)
        winv = win_v[sl]
        w1 = w1_v[sl]
        w2 = w2_v[sl]
        mf = jnp.where(winv >= 0, 1.0, 0.0).astype(jnp.float32)
        wc = jnp.maximum(winv, 0)
        row = jnp.zeros((CHUNK,), jnp.int32)
        u0 = plsc.load_gather(uvt_v, [row, wc])
        v0 = plsc.load_gather(uvt_v, [row + 1, wc])
        u1 = plsc.load_gather(uvt_v, [row + 2, wc])
        v1 = plsc.load_gather(uvt_v, [row + 3, wc])
        u2 = plsc.load_gather(uvt_v, [row + 4, wc])
        v2 = plsc.load_gather(uvt_v, [row + 5, wc])
        w3 = 1.0 - w1 - w2
        pu = (w1 * u0 + w2 * u1) + w3 * u2
        pv = (w1 * v0 + w2 * v1) + w3 * v2
        ix = ((pu + 1.0) * 256.0 - 1.0) * 0.5
        iy = ((pv + 1.0) * 256.0 - 1.0) * 0.5
        ixt = ix.astype(jnp.int32).astype(jnp.float32)
        iyt = iy.astype(jnp.int32).astype(jnp.float32)
        ix0 = ixt - jnp.where(ixt > ix, 1.0, 0.0)
        iy0 = iyt - jnp.where(iyt > iy, 1.0, 0.0)
        wx1 = ix - ix0
        wy1 = iy - iy0
        wx0 = 1.0 - wx1
        wy0 = 1.0 - wy1
        vx0 = jnp.where((ix0 >= 0.0) & (ix0 <= 255.0), 1.0, 0.0)
        vx1 = jnp.where((ix0 >= -1.0) & (ix0 <= 254.0), 1.0, 0.0)
        vy0 = jnp.where((iy0 >= 0.0) & (iy0 <= 255.0), 1.0, 0.0)
        vy1 = jnp.where((iy0 >= -1.0) & (iy0 <= 254.0), 1.0, 0.0)
        xi0 = jnp.minimum(jnp.maximum(ix0, 0.0), 255.0).astype(jnp.int32)
        xi1 = jnp.minimum(jnp.maximum(ix0 + 1.0, 0.0), 255.0).astype(jnp.int32)
        yi0 = jnp.minimum(jnp.maximum(iy0, 0.0), 255.0).astype(jnp.int32)
        yi1 = jnp.minimum(jnp.maximum(iy0 + 1.0, 0.0), 255.0).astype(jnp.int32)
        xi0_v[sl] = xi0
        xi1_v[sl] = xi1
        yi0_v[sl] = yi0
        yi1_v[sl] = yi1
        w00_v[sl] = wx0 * wy0 * (vx0 * vy0) * mf
        w10_v[sl] = wx1 * wy0 * (vx1 * vy0) * mf
        w01_v[sl] = wx0 * wy1 * (vx0 * vy1) * mf
        w11_v[sl] = wx1 * wy1 * (vx1 * vy1) * mf

    for ch, o_hbm in enumerate((r_hbm, g_hbm, b_hbm)):
        pltpu.sync_copy(tex_hbm.at[ch], tex_v)

        @pl.loop(0, PIX_PER_W // CHUNK)
        def _(k):
            sl = pl.ds(k * CHUNK, CHUNK)
            xi0 = xi0_v[sl]
            xi1 = xi1_v[sl]
            yi0 = yi0_v[sl]
            yi1 = yi1_v[sl]
            g00 = plsc.load_gather(tex_v, [yi0, xi0])
            g10 = plsc.load_gather(tex_v, [yi0, xi1])
            g01 = plsc.load_gather(tex_v, [yi1, xi0])
            g11 = plsc.load_gather(tex_v, [yi1, xi1])
            r = ((g00 * w00_v[sl] + g10 * w10_v[sl])
                 + g01 * w01_v[sl]) + g11 * w11_v[sl]
            out_v[sl] = r

        pltpu.sync_copy(out_v, o_hbm.at[pl.ds(base, PIX_PER_W)])


def _sample(win, w1, w2, uvt6, uvmap):
    mesh = plsc.VectorSubcoreMesh(core_axis_name="c", subcore_axis_name="s")
    cp = pltpu.CompilerParams()
    if "needs_layout_passes" in pltpu.CompilerParams.__dataclass_fields__:
        cp = dataclasses.replace(cp, needs_layout_passes=False)
    f = pl.kernel(
        _sampler_kernel,
        mesh=mesh,
        compiler_params=cp,
        out_type=[jax.ShapeDtypeStruct((SIZE * SIZE,), jnp.float32)] * 3,
        scratch_types=[
            pltpu.VMEM((PIX_PER_W,), jnp.int32),
            pltpu.VMEM((PIX_PER_W,), jnp.float32),
            pltpu.VMEM((PIX_PER_W,), jnp.float32),
            pltpu.VMEM((6, NTRI), jnp.float32),
            pltpu.VMEM((PIX_PER_W,), jnp.int32),
            pltpu.VMEM((PIX_PER_W,), jnp.int32),
            pltpu.VMEM((PIX_PER_W,), jnp.int32),
            pltpu.VMEM((PIX_PER_W,), jnp.int32),
            pltpu.VMEM((PIX_PER_W,), jnp.float32),
            pltpu.VMEM((PIX_PER_W,), jnp.float32),
            pltpu.VMEM((PIX_PER_W,), jnp.float32),
            pltpu.VMEM((PIX_PER_W,), jnp.float32),
            pltpu.VMEM((SIZE, SIZE), jnp.float32),
            pltpu.VMEM((PIX_PER_W,), jnp.float32),
            pltpu.SemaphoreType.DMA,
        ],
    )
    return f(win, w1, w2, uvt6, uvmap)


def kernel(vertices, faces, uv, uvfaces, uvmap):
    size = SIZE
    lin = jnp.linspace(-1.0, 1.0, size, dtype=jnp.float32)
    px = lin.reshape(1, size)
    py = lin[::-1].reshape(size, 1)

    tris = vertices[faces]
    uv2 = uv * 2.0 - 1.0
    uvt = uv2[uvfaces]
    zmin = jnp.min(vertices[:, 2]).reshape(1, 1)

    Ax, Ay, Az = tris[:, 0, 0], tris[:, 0, 1], tris[:, 0, 2]
    Bx, By, Bz = tris[:, 1, 0], tris[:, 1, 1], tris[:, 1, 2]
    Cx, Cy, Cz = tris[:, 2, 0], tris[:, 2, 1], tris[:, 2, 2]
    nz = (Bx - Ax) * (Cy - Ay) - (By - Ay) * (Cx - Ax)
    valid = (nz > 0) & (nz >= 1e-9)
    ws = jnp.where(jnp.abs(nz) < 1e-12, 1.0, nz)
    invw = 1.0 / ws
    consts = jnp.stack([
        Bx, By, Ay - By, Ax - Bx,
        Cx, Cy, By - Cy, Bx - Cx,
        Ax, Ay, Cy - Ay, Cx - Ax,
        Az, Bz, Cz, invw,
    ], axis=0)
    consts = jnp.where(valid[None, :], consts, 0.0)

    win, w1, w2, alpha = _rasterize(consts, px, py, zmin)

    uvt6 = jnp.transpose(uvt, (1, 2, 0)).reshape(6, NTRI)
    r, g, b = _sample(win.reshape(-1), w1.reshape(-1), w2.reshape(-1),
                      uvt6, uvmap)
    rgb = jnp.stack([r, g, b], axis=0).reshape(3, size, size)
    return jnp.concatenate([rgb, alpha[None]], axis=0)


# speedup vs baseline: 5311.7933x; 2.2149x over previous
import dataclasses
import functools

import jax
import jax.numpy as jnp
from jax import lax
from jax.experimental import pallas as pl
from jax.experimental.pallas import tpu as pltpu
from jax.experimental.pallas import tpu_sc as plsc

SIZE = 256
NTRI = 256
ROWS_PER_TILE = 16
NTILES = SIZE // ROWS_PER_TILE
NWORKERS = 32
PIX_PER_W = SIZE * SIZE // NWORKERS
CHUNK = 16
NROWS = 22
PAD = NTRI + CHUNK


def _sc_compiler_params():
    cp = pltpu.CompilerParams()
    if "needs_layout_passes" in pltpu.CompilerParams.__dataclass_fields__:
        cp = dataclasses.replace(cp, needs_layout_passes=False)
    return cp


def _sc_mesh():
    return plsc.VectorSubcoreMesh(core_axis_name="c", subcore_axis_name="s")


def _prep_kernel(vert_hbm, faces_hbm, uv_hbm, uvf_hbm, out_hbm,
                 vert_v, faces_v, uv_v, uvf_v, meta_v, sem, *rows_v):
    wid = lax.axis_index("c") * 16 + lax.axis_index("s")

    @pl.when(wid == 0)
    def _():
        pltpu.sync_copy(vert_hbm, vert_v)
        pltpu.sync_copy(faces_hbm, faces_v)
        pltpu.sync_copy(uv_hbm, uv_v)
        pltpu.sync_copy(uvf_hbm, uvf_v)

        iota = lax.iota(jnp.int32, CHUNK)

        def zbody(k, acc):
            idx = jnp.minimum(k * CHUNK + iota, 199) * 3 + 2
            vz = plsc.load_gather(vert_v, [idx])
            return jnp.minimum(acc, vz)

        zv = lax.fori_loop(0, 13, zbody,
                           jnp.full((CHUNK,), jnp.inf, jnp.float32))

        def body(k, cnt):
            tid3 = (k * CHUNK + iota) * 3
            fa = plsc.load_gather(faces_v, [tid3]) * 3
            fb = plsc.load_gather(faces_v, [tid3 + 1]) * 3
            fc = plsc.load_gather(faces_v, [tid3 + 2]) * 3
            Ax = plsc.load_gather(vert_v, [fa])
            Ay = plsc.load_gather(vert_v, [fa + 1])
            Az = plsc.load_gather(vert_v, [fa + 2])
            Bx = plsc.load_gather(vert_v, [fb])
            By = plsc.load_gather(vert_v, [fb + 1])
            Bz = plsc.load_gather(vert_v, [fb + 2])
            Cx = plsc.load_gather(vert_v, [fc])
            Cy = plsc.load_gather(vert_v, [fc + 1])
            Cz = plsc.load_gather(vert_v, [fc + 2])
            ga = plsc.load_gather(uvf_v, [tid3]) * 2
            gb = plsc.load_gather(uvf_v, [tid3 + 1]) * 2
            gc = plsc.load_gather(uvf_v, [tid3 + 2]) * 2
            u0 = plsc.load_gather(uv_v, [ga]) * 2.0 - 1.0
            v0 = plsc.load_gather(uv_v, [ga + 1]) * 2.0 - 1.0
            u1 = plsc.load_gather(uv_v, [gb]) * 2.0 - 1.0
            v1 = plsc.load_gather(uv_v, [gb + 1]) * 2.0 - 1.0
            u2 = plsc.load_gather(uv_v, [gc]) * 2.0 - 1.0
            v2 = plsc.load_gather(uv_v, [gc + 1]) * 2.0 - 1.0

            nz = (Bx - Ax) * (Cy - Ay) - (By - Ay) * (Cx - Ax)
            valid = (nz > 0) & (nz >= 1e-9)
            ws = jnp.where(jnp.abs(nz) < 1e-12, 1.0, nz)
            invw = 1.0 / ws

            vals = (Bx, By, Ay - By, Ax - Bx,
                    Cx, Cy, By - Cy, Bx - Cx,
                    Ax, Ay, Cy - Ay, Cx - Ax,
                    Az, Bz, Cz, invw,
                    u0, v0, u1, v1, u2, v2)
            pos = cnt + plsc.cumsum(valid.astype(jnp.int32)) - 1
            posc = jnp.maximum(pos, 0)
            for r, x in enumerate(vals):
                plsc.store_scatter(rows_v[r], [posc], x, mask=valid)
            return cnt + plsc.all_reduce_population_count(valid)

        cnt = lax.fori_loop(0, NTRI // CHUNK, body,
                            jnp.zeros((CHUNK,), jnp.int32))

        zmin = jnp.min(zv)
        meta_v[pl.ds(0, CHUNK)] = jnp.full((CHUNK,), zmin, jnp.float32)
        meta_v[pl.ds(CHUNK, CHUNK)] = cnt.astype(jnp.float32)
        for r in range(NROWS):
            pltpu.sync_copy(rows_v[r].at[pl.ds(0, NTRI)],
                            out_hbm.at[pl.ds(r * NTRI, NTRI)])
        pltpu.sync_copy(meta_v, out_hbm.at[pl.ds(NROWS * NTRI, 2 * CHUNK)])


def _prep(vertices, faces, uv, uvfaces):
    f = pl.kernel(
        _prep_kernel,
        mesh=_sc_mesh(),
        compiler_params=_sc_compiler_params(),
        out_type=jax.ShapeDtypeStruct((NROWS * NTRI + 2 * CHUNK,),
                                      jnp.float32),
        scratch_types=[
            pltpu.VMEM((600,), jnp.float32),
            pltpu.VMEM((NTRI * 3,), jnp.int32),
            pltpu.VMEM((600,), jnp.float32),
            pltpu.VMEM((NTRI * 3,), jnp.int32),
            pltpu.VMEM((2 * CHUNK,), jnp.float32),
            pltpu.SemaphoreType.DMA,
        ] + [pltpu.VMEM((PAD,), jnp.float32)] * NROWS,
    )
    return f(vertices.reshape(-1), faces.reshape(-1),
             uv.reshape(-1), uvfaces.reshape(-1))


def _raster_body(consts_ref, px_ref, py_ref, zmin_ref, count_ref,
                 win_ref, w1_ref, w2_ref, a_ref):
    pxb = jnp.broadcast_to(px_ref[...], (ROWS_PER_TILE, SIZE))
    pyb = jnp.broadcast_to(py_ref[...], (ROWS_PER_TILE, SIZE))
    zmin = zmin_ref[0, 0]
    zbuf0 = jnp.full((ROWS_PER_TILE, SIZE), zmin, jnp.float32)
    win0 = jnp.full((ROWS_PER_TILE, SIZE), -1, jnp.int32)
    w10 = jnp.zeros((ROWS_PER_TILE, SIZE), jnp.float32)
    w20 = jnp.zeros((ROWS_PER_TILE, SIZE), jnp.float32)

    def body(i, carry):
        zbuf, win, w1s, w2s = carry
        Bx = consts_ref[0, i]
        By = consts_ref[1, i]
        dABy = consts_ref[2, i]
        dABx = consts_ref[3, i]
        Cx = consts_ref[4, i]
        Cy = consts_ref[5, i]
        dBCy = consts_ref[6, i]
        dBCx = consts_ref[7, i]
        Ax = consts_ref[8, i]
        Ay = consts_ref[9, i]
        dCAy = consts_ref[10, i]
        dCAx = consts_ref[11, i]
        Az = consts_ref[12, i]
        Bz = consts_ref[13, i]
        Cz = consts_ref[14, i]
        invw = consts_ref[15, i]
        pAB = (pxb - Bx) * dABy - (pyb - By) * dABx
        pCB = (pxb - Cx) * dBCy - (pyb - Cy) * dBCx
        pCA = (pxb - Ax) * dCAy - (pyb - Ay) * dCAx
        inside = (jnp.maximum(pAB, 0.0) * jnp.maximum(pCB, 0.0)
                  * jnp.maximum(pCA, 0.0)) > 0
        w1 = pCB * invw
        w2 = pCA * invw
        w3 = 1.0 - w1 - w2
        z = (w1 * Az + w2 * Bz) + w3 * Cz
        mask = inside & (z >= zbuf)
        zbuf = jnp.where(mask, z, zbuf)
        win = jnp.where(mask, i, win)
        w1s = jnp.where(mask, w1, w1s)
        w2s = jnp.where(mask, w2, w2s)
        return (zbuf, win, w1s, w2s)

    nt = count_ref[0, 0]
    zbuf, win, w1s, w2s = lax.fori_loop(0, nt, body,
                                        (zbuf0, win0, w10, w20))
    win_ref[...] = win
    w1_ref[...] = w1s
    w2_ref[...] = w2s
    a_ref[...] = jnp.where(win >= 0, 1.0, 0.0).astype(jnp.float32)


def _rasterize(consts, px, py, zmin, count):
    out = pl.pallas_call(
        _raster_body,
        grid=(NTILES,),
        in_specs=[
            pl.BlockSpec(memory_space=pltpu.SMEM),
            pl.BlockSpec((1, SIZE), lambda i: (0, 0)),
            pl.BlockSpec((ROWS_PER_TILE, 1), lambda i: (i, 0)),
            pl.BlockSpec(memory_space=pltpu.SMEM),
            pl.BlockSpec(memory_space=pltpu.SMEM),
        ],
        out_specs=[
            pl.BlockSpec((ROWS_PER_TILE, SIZE), lambda i: (i, 0)),
            pl.BlockSpec((ROWS_PER_TILE, SIZE), lambda i: (i, 0)),
            pl.BlockSpec((ROWS_PER_TILE, SIZE), lambda i: (i, 0)),
            pl.BlockSpec((ROWS_PER_TILE, SIZE), lambda i: (i, 0)),
        ],
        out_shape=[
            jax.ShapeDtypeStruct((SIZE, SIZE), jnp.int32),
            jax.ShapeDtypeStruct((SIZE, SIZE), jnp.float32),
            jax.ShapeDtypeStruct((SIZE, SIZE), jnp.float32),
            jax.ShapeDtypeStruct((SIZE, SIZE), jnp.float32),
        ],
    )(consts, px, py, zmin, count)
    return out


def _sampler_kernel(win_hbm, w1_hbm, w2_hbm, uvt_hbm, tex_hbm,
                    r_hbm, g_hbm, b_hbm,
                    win_v, w1_v, w2_v, uvt_v,
                    xi0_v, yi0_v,
                    wx0_v, wx1_v, wy0_v, wy1_v, tex_v, out_v, sem):
    wid = lax.axis_index("c") * 16 + lax.axis_index("s")
    base = wid * PIX_PER_W

    pltpu.sync_copy(win_hbm.at[pl.ds(base, PIX_PER_W)], win_v)
    pltpu.sync_copy(w1_hbm.at[pl.ds(base, PIX_PER_W)], w1_v)
    pltpu.sync_copy(w2_hbm.at[pl.ds(base, PIX_PER_W)], w2_v)
    pltpu.sync_copy(uvt_hbm, uvt_v)

    @pl.loop(0, PIX_PER_W // CHUNK)
    def _(k):
        sl = pl.ds(k * CHUNK, CHUNK)
        winv = win_v[sl]
        w1 = w1_v[sl]
        w2 = w2_v[sl]
        mf = jnp.where(winv >= 0, 1.0, 0.0).astype(jnp.float32)
        wc = jnp.maximum(winv, 0)
        row = jnp.zeros((CHUNK,), jnp.int32)
        u0 = plsc.load_gather(uvt_v, [row, wc])
        v0 = plsc.load_gather(uvt_v, [row + 1, wc])
        u1 = plsc.load_gather(uvt_v, [row + 2, wc])
        v1 = plsc.load_gather(uvt_v, [row + 3, wc])
        u2 = plsc.load_gather(uvt_v, [row + 4, wc])
        v2 = plsc.load_gather(uvt_v, [row + 5, wc])
        w3 = 1.0 - w1 - w2
        pu = (w1 * u0 + w2 * u1) + w3 * u2
        pv = (w1 * v0 + w2 * v1) + w3 * v2
        ix = ((pu + 1.0) * 256.0 - 1.0) * 0.5
        iy = ((pv + 1.0) * 256.0 - 1.0) * 0.5
        ixt = ix.astype(jnp.int32).astype(jnp.float32)
        iyt = iy.astype(jnp.int32).astype(jnp.float32)
        ix0 = ixt - jnp.where(ixt > ix, 1.0, 0.0)
        iy0 = iyt - jnp.where(iyt > iy, 1.0, 0.0)
        wx1 = ix - ix0
        wy1 = iy - iy0
        wx0 = 1.0 - wx1
        wy0 = 1.0 - wy1
        vx0 = jnp.where((ix0 >= 0.0) & (ix0 <= 255.0), 1.0, 0.0)
        vx1 = jnp.where((ix0 >= -1.0) & (ix0 <= 254.0), 1.0, 0.0)
        vy0 = jnp.where((iy0 >= 0.0) & (iy0 <= 255.0), 1.0, 0.0)
        vy1 = jnp.where((iy0 >= -1.0) & (iy0 <= 254.0), 1.0, 0.0)
        xi0 = jnp.minimum(jnp.maximum(ix0, 0.0), 255.0).astype(jnp.int32)
        yi0 = jnp.minimum(jnp.maximum(iy0, 0.0), 255.0).astype(jnp.int32)
        xi0_v[sl] = xi0
        yi0_v[sl] = yi0
        wx0_v[sl] = wx0 * vx0
        wx1_v[sl] = wx1 * vx1
        wy0_v[sl] = wy0 * vy0 * mf
        wy1_v[sl] = wy1 * vy1 * mf

    for ch, o_hbm in enumerate((r_hbm, g_hbm, b_hbm)):
        pltpu.sync_copy(tex_hbm.at[ch], tex_v)

        @pl.loop(0, PIX_PER_W // CHUNK)
        def _(k):
            sl = pl.ds(k * CHUNK, CHUNK)
            xi0 = xi0_v[sl]
            yi0 = yi0_v[sl]
            xi1 = jnp.minimum(xi0 + 1, 255)
            yi1 = jnp.minimum(yi0 + 1, 255)
            wx0 = wx0_v[sl]
            wx1 = wx1_v[sl]
            wy0 = wy0_v[sl]
            wy1 = wy1_v[sl]
            g00 = plsc.load_gather(tex_v, [yi0, xi0])
            g10 = plsc.load_gather(tex_v, [yi0, xi1])
            g01 = plsc.load_gather(tex_v, [yi1, xi0])
            g11 = plsc.load_gather(tex_v, [yi1, xi1])
            r = ((g00 * (wx0 * wy0) + g10 * (wx1 * wy0))
                 + g01 * (wx0 * wy1)) + g11 * (wx1 * wy1)
            out_v[sl] = r

        pltpu.sync_copy(out_v, o_hbm.at[pl.ds(base, PIX_PER_W)])


def _sample(win, w1, w2, uvt6, uvmap):
    f = pl.kernel(
        _sampler_kernel,
        mesh=_sc_mesh(),
        compiler_params=_sc_compiler_params(),
        out_type=[jax.ShapeDtypeStruct((SIZE * SIZE,), jnp.float32)] * 3,
        scratch_types=[
            pltpu.VMEM((PIX_PER_W,), jnp.int32),
            pltpu.VMEM((PIX_PER_W,), jnp.float32),
            pltpu.VMEM((PIX_PER_W,), jnp.float32),
            pltpu.VMEM((6, NTRI), jnp.float32),
            pltpu.VMEM((PIX_PER_W,), jnp.int32),
            pltpu.VMEM((PIX_PER_W,), jnp.int32),
            pltpu.VMEM((PIX_PER_W,), jnp.float32),
            pltpu.VMEM((PIX_PER_W,), jnp.float32),
            pltpu.VMEM((PIX_PER_W,), jnp.float32),
            pltpu.VMEM((PIX_PER_W,), jnp.float32),
            pltpu.VMEM((SIZE, SIZE), jnp.float32),
            pltpu.VMEM((PIX_PER_W,), jnp.float32),
            pltpu.SemaphoreType.DMA,
        ],
    )
    return f(win, w1, w2, uvt6, uvmap)


def kernel(vertices, faces, uv, uvfaces, uvmap):
    size = SIZE
    lin = jnp.linspace(-1.0, 1.0, size, dtype=jnp.float32)
    px = lin.reshape(1, size)
    py = lin[::-1].reshape(size, 1)

    prep = _prep(vertices, faces, uv, uvfaces)
    consts = prep[:16 * NTRI].reshape(16, NTRI)
    uvt6 = prep[16 * NTRI:NROWS * NTRI].reshape(6, NTRI)
    zmin = prep[NROWS * NTRI].reshape(1, 1)
    count = prep[NROWS * NTRI + CHUNK].astype(jnp.int32).reshape(1, 1)

    win, w1, w2, alpha = _rasterize(consts, px, py, zmin, count)

    r, g, b = _sample(win.reshape(-1), w1.reshape(-1), w2.reshape(-1),
                      uvt6, uvmap)
    rgb = jnp.stack([r, g, b], axis=0).reshape(3, size, size)
    return jnp.concatenate([rgb, alpha[None]], axis=0)
